# Initial kernel scaffold; baseline (speedup 1.0000x reference)
import jax
import jax.numpy as jnp
from jax.experimental import pallas as pl


def kernel(x, adj_indices, adj_values, W):
    raise NotImplementedError("write your pallas kernel here")


# speedup vs baseline: 3.4453x; 3.4453x over previous
import functools

import jax
import jax.numpy as jnp
from jax import lax
from jax.experimental import pallas as pl
from jax.experimental.pallas import tpu as pltpu
from jax.experimental.pallas import tpu_sc as plsc

N = 10000
E = 160000
D_IN = 256
D_OUT = 256

NC = 2
NS = 16
L = 16
C = 128
DH = D_OUT // NC
NCH = E // C
WB_ROWS = 624
WB_EXTRA = N - NS * WB_ROWS

_MM_BLOCK = 400


def _matmul_kernel(x_ref, w_ref, o_ref):
    o_ref[...] = jnp.dot(x_ref[...], w_ref[...],
                         preferred_element_type=jnp.float32)


def _matmul(x, w):
    return pl.pallas_call(
        _matmul_kernel,
        grid=(N // _MM_BLOCK,),
        in_specs=[
            pl.BlockSpec((_MM_BLOCK, D_IN), lambda i: (i, 0)),
            pl.BlockSpec((D_IN, D_OUT), lambda i: (0, 0)),
        ],
        out_specs=pl.BlockSpec((_MM_BLOCK, D_OUT), lambda i: (i, 0)),
        out_shape=jax.ShapeDtypeStruct((N, D_OUT), jnp.float32),
    )(x, w)


def _sc_kernel(pre_hbm, rows_hbm, cols_hbm, vals_hbm, out_hbm,
               idx_v, rid_v, val_v, gbuf, acc, sem):
    c = lax.axis_index("c")
    s = lax.axis_index("s")

    zeros = jnp.zeros((L,), jnp.float32)

    def zero_row(i, _):
        for f in range(DH // L):
            gbuf[i, pl.ds(f * L, L)] = zeros
        return 0

    lax.fori_loop(0, C, zero_row, 0)
    r0 = pl.multiple_of(s * WB_ROWS, 8)
    for j in range(4):
        pltpu.sync_copy(gbuf.at[pl.ds(0, C)],
                        acc.at[pl.ds(r0 + j * C, C)])
    pltpu.sync_copy(gbuf.at[pl.ds(0, WB_ROWS - 4 * C)],
                    acc.at[pl.ds(r0 + 4 * C, WB_ROWS - 4 * C)])

    @pl.when(s == NS - 1)
    def _():
        pltpu.sync_copy(gbuf.at[pl.ds(0, WB_EXTRA)],
                        acc.at[pl.ds(NS * WB_ROWS, WB_EXTRA)])

    plsc.subcore_barrier()

    nfull = NCH // NS
    rem = NCH % NS
    nc_mine = nfull + jnp.where(s < rem, 1, 0)

    def chunk_body(t, _):
        base = (t * NS + s) * C
        pltpu.sync_copy(cols_hbm.at[pl.ds(base, C)], idx_v)
        pltpu.sync_copy(rows_hbm.at[pl.ds(base, C)], rid_v)
        pltpu.sync_copy(vals_hbm.at[pl.ds(base, C)], val_v)
        for i in range(C // L):
            sl = pl.ds(i * L, L)
            idx_v[sl] = idx_v[sl] * 2 + c
        pltpu.async_copy(pre_hbm.at[idx_v], gbuf, sem).wait()

        def scale_grp(g, _):
            e0 = pl.multiple_of(g * L, L)
            v16 = val_v[pl.ds(e0, L)]
            for j in range(L):
                v = v16[j]
                for f in range(DH // L):
                    sl = pl.ds(f * L, L)
                    gbuf[e0 + j, sl] = gbuf[e0 + j, sl] * v
            return 0

        lax.fori_loop(0, C // L, scale_grp, 0)
        pltpu.sync_copy(gbuf, acc.at[rid_v], add=True)
        return 0

    lax.fori_loop(0, nc_mine, chunk_body, 0)
    plsc.subcore_barrier()

    pltpu.sync_copy(
        acc.at[pl.ds(r0, WB_ROWS)],
        out_hbm.at[pl.ds(r0, WB_ROWS), pl.ds(c * DH, DH)])

    @pl.when(s == NS - 1)
    def _():
        pltpu.sync_copy(
            acc.at[pl.ds(NS * WB_ROWS, WB_EXTRA)],
            out_hbm.at[pl.ds(NS * WB_ROWS, WB_EXTRA), pl.ds(c * DH, DH)])


def _sc_scatter(pre2, rows, cols, vals):
    mesh = plsc.VectorSubcoreMesh(core_axis_name="c", subcore_axis_name="s")
    f = pl.kernel(
        _sc_kernel,
        out_type=jax.ShapeDtypeStruct((N, D_OUT), jnp.float32),
        mesh=mesh,
        scratch_types=[
            pltpu.VMEM((C,), jnp.int32),
            pltpu.VMEM((C,), jnp.int32),
            pltpu.VMEM((C,), jnp.float32),
            pltpu.VMEM((C, DH), jnp.float32),
            pltpu.VMEM_SHARED((N, DH), jnp.float32),
            pltpu.SemaphoreType.DMA,
        ],
    )
    return f(pre2, rows, cols, vals)


@jax.jit
def kernel(x, adj_indices, adj_values, W):
    pre = _matmul(x, W)
    pre2 = pre.reshape(2 * N, DH)
    rows = adj_indices[0]
    cols = adj_indices[1]
    return _sc_scatter(pre2, rows, cols, adj_values)


# speedup vs baseline: 7.0617x; 2.0497x over previous
import jax
import jax.numpy as jnp
from jax import lax
from jax.experimental import pallas as pl
from jax.experimental.pallas import tpu as pltpu
from jax.experimental.pallas import tpu_sc as plsc

N = 10000
E = 160000
D_IN = 256
D_OUT = 256

NC = 2
NS = 16
L = 16
C = 80
DH = D_OUT // NC
CPT = E // (NS * C)
NB = 4
NR = 8
WB_ROWS = 624
WB_EXTRA = N - NS * WB_ROWS

_MM_BLOCK = 400


def _matmul_kernel(x_ref, w_ref, o_ref):
    o_ref[...] = jnp.dot(x_ref[...], w_ref[...],
                         preferred_element_type=jnp.float32)


def _matmul(x, w):
    return pl.pallas_call(
        _matmul_kernel,
        grid=(N // _MM_BLOCK,),
        in_specs=[
            pl.BlockSpec((_MM_BLOCK, D_IN), lambda i: (i, 0)),
            pl.BlockSpec((D_IN, D_OUT), lambda i: (0, 0)),
        ],
        out_specs=pl.BlockSpec((_MM_BLOCK, D_OUT), lambda i: (i, 0)),
        out_shape=jax.ShapeDtypeStruct((N, D_OUT), jnp.float32),
    )(x, w)


def _sc_kernel(pre_hbm, rows_hbm, cols_hbm, vals_hbm, out_hbm,
               idx2d, rid2d, val2, gbuf, acc,
               st0, st1, st2, st3, sg0, sg1, sg2, sg3, ss0, ss1, ss2, ss3):
    st = [st0, st1, st2, st3]
    sg = [sg0, sg1, sg2, sg3]
    ss = [ss0, ss1, ss2, ss3]
    c = lax.axis_index("c")
    s = lax.axis_index("s")
    tile_base = s * CPT

    def ebase(t):
        return pl.multiple_of((tile_base + t) * C, 16)

    def stage(t, p):
        r4, r8 = p % NB, p % NR
        pltpu.async_copy(cols_hbm.at[pl.ds(ebase(t), C)], idx2d.at[r8],
                         st[r4])
        pltpu.async_copy(rows_hbm.at[pl.ds(ebase(t), C)], rid2d.at[r8],
                         st[r4])
        pltpu.async_copy(vals_hbm.at[pl.ds(ebase(t), C)], val2.at[r8],
                         st[r4])

    def prep(t, p):
        r4, r8, b = p % NB, p % NR, p % NB
        pltpu.make_async_copy(cols_hbm.at[pl.ds(0, C)], idx2d.at[r8],
                              st[r4]).wait()
        pltpu.make_async_copy(rows_hbm.at[pl.ds(0, C)], rid2d.at[r8],
                              st[r4]).wait()
        pltpu.make_async_copy(vals_hbm.at[pl.ds(0, C)], val2.at[r8],
                              st[r4]).wait()
        for f in range(C // L):
            sl = pl.ds(f * L, L)
            idx2d[r8, sl] = idx2d[r8, sl] * 2 + c
        pltpu.async_copy(pre_hbm.at[idx2d.at[r8]], gbuf.at[b], sg[b])

    def proc(t, p):
        r8, b = p % NR, p % NB
        pltpu.make_async_copy(pre_hbm.at[idx2d.at[0]], gbuf.at[b],
                              sg[b]).wait()

        def grp(g, _):
            o = pl.multiple_of(g * L, L)
            v16 = val2[r8, pl.ds(o, L)]
            for j in range(L):
                v = v16[j]
                for f in range(DH // L):
                    sl = pl.ds(f * L, L)
                    gbuf[b, o + j, sl] = gbuf[b, o + j, sl] * v
            return 0

        lax.fori_loop(0, C // L, grp, 0)
        pltpu.async_copy(gbuf.at[b], acc.at[rid2d.at[r8]], ss[b], add=True)

    def scatter_wait(b):
        pltpu.make_async_copy(gbuf.at[b], acc.at[rid2d.at[0]], ss[b]).wait()

    zeros = jnp.zeros((L,), jnp.float32)

    def zero_row(i, _):
        for f in range(DH // L):
            gbuf[NB - 1, i, pl.ds(f * L, L)] = zeros
        return 0

    lax.fori_loop(0, C, zero_row, 0)
    r0 = pl.multiple_of(s * WB_ROWS, 8)
    for j in range(7):
        pltpu.sync_copy(gbuf.at[NB - 1, pl.ds(0, C)],
                        acc.at[pl.ds(r0 + j * C, C)])
    pltpu.sync_copy(gbuf.at[NB - 1, pl.ds(0, WB_ROWS - 7 * C)],
                    acc.at[pl.ds(r0 + 7 * C, WB_ROWS - 7 * C)])

    @pl.when(s == NS - 1)
    def _():
        pltpu.sync_copy(gbuf.at[NB - 1, pl.ds(0, WB_EXTRA)],
                        acc.at[pl.ds(NS * WB_ROWS, WB_EXTRA)])

    plsc.subcore_barrier()

    for t in range(NB):
        stage(t, t)
    prep(0, 0)
    prep(1, 1)

    proc(0, 0)
    prep(2, 2)
    stage(4, 4)
    proc(1, 1)
    prep(3, 3)
    stage(5, 5)

    def pipe_body(u, _):
        for k in range(NR):
            t = u * NR + 2 + k
            ph = (2 + k) % NR
            proc(t, ph)
            scatter_wait((ph + 2) % NB)
            prep(t + 2, (ph + 2) % NR)

            @pl.when(t + NB < CPT)
            def _():
                stage(t + NB, (ph + NB) % NR)
        return 0

    lax.fori_loop(0, (CPT - 5) // NR, pipe_body, 0)
    proc(CPT - 3, (CPT - 3) % NR)
    scatter_wait((CPT - 1) % NB)
    prep(CPT - 1, (CPT - 1) % NR)
    proc(CPT - 2, (CPT - 2) % NR)
    proc(CPT - 1, (CPT - 1) % NR)
    for b in range(NB):
        scatter_wait(b)

    plsc.subcore_barrier()

    pltpu.sync_copy(
        acc.at[pl.ds(r0, WB_ROWS)],
        out_hbm.at[pl.ds(r0, WB_ROWS), pl.ds(c * DH, DH)])

    @pl.when(s == NS - 1)
    def _():
        pltpu.sync_copy(
            acc.at[pl.ds(NS * WB_ROWS, WB_EXTRA)],
            out_hbm.at[pl.ds(NS * WB_ROWS, WB_EXTRA), pl.ds(c * DH, DH)])


def _sc_scatter(pre2, rows, cols, vals):
    mesh = plsc.VectorSubcoreMesh(core_axis_name="c", subcore_axis_name="s")
    f = pl.kernel(
        _sc_kernel,
        out_type=jax.ShapeDtypeStruct((N, D_OUT), jnp.float32),
        mesh=mesh,
        scratch_types=(
            [
                pltpu.VMEM((NR, C), jnp.int32),
                pltpu.VMEM((NR, C), jnp.int32),
                pltpu.VMEM((NR, C), jnp.float32),
                pltpu.VMEM((NB, C, DH), jnp.float32),
                pltpu.VMEM_SHARED((N, DH), jnp.float32),
            ]
            + [pltpu.SemaphoreType.DMA] * 12
        ),
    )
    return f(pre2, rows, cols, vals)


@jax.jit
def kernel(x, adj_indices, adj_values, W):
    pre = _matmul(x, W)
    pre2 = pre.reshape(2 * N, DH)
    rows = adj_indices[0]
    cols = adj_indices[1]
    return _sc_scatter(pre2, rows, cols, adj_values)
